# baseline (speedup 1.0000x reference)
import functools

import jax
import jax.numpy as jnp
from jax import lax
from jax.experimental import pallas as pl
from jax.experimental.pallas import tpu as pltpu
from jax.experimental.pallas import tpu_sc as plsc

B, S, V, H = 4, 2048, 128100, 1024
NT = B * S
LN_EPS = 1e-07

_info = plsc.get_sparse_core_info()
NC, NS = _info.num_cores, _info.num_subcores
NW = NC * NS
NSLICE = 4
QS = S // NSLICE
TS = B * QS
WPB = NW // B
T_PER_W = TS // NW
CHUNK = 32
N_CHUNKS = T_PER_W // CHUNK


def _sc_gather_slice(idx_slice, table):
    mesh = plsc.VectorSubcoreMesh(core_axis_name="c", subcore_axis_name="s")

    @functools.partial(
        pl.kernel,
        mesh=mesh,
        out_type=jax.ShapeDtypeStruct((TS, H), jnp.float32),
        scratch_types=[
            pltpu.VMEM((T_PER_W,), jnp.int32),
            pltpu.VMEM((N_CHUNKS, CHUNK, H), jnp.float32),
            pltpu.SemaphoreType.DMA,
            pltpu.SemaphoreType.DMA,
            pltpu.SemaphoreType.DMA,
            pltpu.SemaphoreType.DMA,
        ],
    )
    def k(idx_hbm, table_hbm, out_hbm, idx_v, rows_v, g0, g1, w0, w1):
        wid = lax.axis_index("s") * NC + lax.axis_index("c")
        gsem = (g0, g1)
        wsem = (w0, w1)
        pltpu.sync_copy(idx_hbm.at[pl.ds(wid * T_PER_W, T_PER_W)], idx_v)
        gh = [pltpu.async_copy(
                  table_hbm.at[idx_v.at[pl.ds(c * CHUNK, CHUNK)]],
                  rows_v.at[c], gsem[c])
              for c in range(N_CHUNKS)]
        wh = []
        for c in range(N_CHUNKS):
            gh[c].wait()
            wh.append(pltpu.async_copy(
                rows_v.at[c],
                out_hbm.at[pl.ds(wid * T_PER_W + c * CHUNK, CHUNK)],
                wsem[c]))
        for h in wh:
            h.wait()

    return k(idx_slice, table)


ROWS_BLK = 512
PB = QS // ROWS_BLK


def _ln_body(g_ref, p_ref, s_ref, b_ref, *rest):
    o_ref = rest[-1]
    x = g_ref[...] + p_ref[...]
    mean = jnp.mean(x, axis=-1, keepdims=True)
    var = jnp.mean(jnp.square(x - mean), axis=-1, keepdims=True)
    normed = (x - mean) * lax.rsqrt(var + LN_EPS)
    o_ref[...] = normed * s_ref[...] + b_ref[...]


def _tc_add_ln_slice(gathered, pos, scale, bias, buf, s):
    operands = [gathered, pos, scale, bias]
    in_specs = [
        pl.BlockSpec((ROWS_BLK, H), lambda i, j: (j * PB + i, 0)),
        pl.BlockSpec((ROWS_BLK, H), lambda i, j, s=s: (s * PB + i, 0)),
        pl.BlockSpec((1, H), lambda i, j: (0, 0)),
        pl.BlockSpec((1, H), lambda i, j: (0, 0)),
    ]
    aliases = {}
    if buf is not None:
        operands.append(buf)
        in_specs.append(pl.BlockSpec(memory_space=pl.ANY))
        aliases = {4: 0}
    return pl.pallas_call(
        _ln_body,
        grid=(PB, B),
        in_specs=in_specs,
        out_specs=pl.BlockSpec(
            (ROWS_BLK, H),
            lambda i, j, s=s: (j * (S // ROWS_BLK) + s * PB + i, 0)),
        out_shape=jax.ShapeDtypeStruct((NT, H), jnp.float32),
        input_output_aliases=aliases,
    )(*operands)


def kernel(input_ids, word_embeddings, position_embeddings, ln_scale, ln_bias):
    ids = input_ids.astype(jnp.int32)
    scale2 = ln_scale.reshape(1, H)
    bias2 = ln_bias.reshape(1, H)
    gathered = [
        _sc_gather_slice(ids[:, s * QS:(s + 1) * QS].reshape(TS),
                         word_embeddings)
        for s in range(NSLICE)
    ]
    buf = None
    for s in range(NSLICE):
        buf = _tc_add_ln_slice(gathered[s], position_embeddings,
                               scale2, bias2, buf, s)
    return buf.reshape(B, S, H)
